# baseline (speedup 1.0000x reference)
import functools

import jax
import jax.numpy as jnp
from jax.experimental import pallas as pl
from jax.experimental.pallas import tpu as pltpu


def _stg_kernel(x_ref, w_ref, b_ref, out_ref, scores_ref,
                selk_ref, selx_ref, *, topk: int):
    topk_f = jnp.float32(topk)

    keys = selk_ref[...]

    def bit_step(i, mag):
        cand = mag + (jnp.int32(1) << (29 - i))
        cnt = jnp.sum((keys >= cand).astype(jnp.float32), axis=1,
                      keepdims=True)
        return jnp.where(cnt >= topk_f, cand, mag)

    mag = jax.lax.fori_loop(
        0, 30, bit_step, jnp.zeros((keys.shape[0], 1), jnp.int32))
    out_ref[...] = jnp.where(keys >= mag, selx_ref[...], 0.0)

    prod = jax.lax.dot_general(
        x_ref[...].astype(jnp.bfloat16), w_ref[...],
        dimension_numbers=(((1,), (1,)), ((), ())),
        preferred_element_type=jnp.float32)
    scores = jax.nn.sigmoid(prod + b_ref[...])
    scores_ref[...] = scores
    selk_ref[...] = jax.lax.bitcast_convert_type(scores, jnp.int32)
    selx_ref[...] = x_ref[...]


def kernel(x, W, b):
    m, kdim = x.shape
    n = W.shape[0]
    topk = max(1, int(0.3 * n))
    bm = min(128, m)
    nm = m // bm
    grid = (nm + 1,)
    last = nm - 1

    masked, scores = pl.pallas_call(
        functools.partial(_stg_kernel, topk=topk),
        grid=grid,
        in_specs=[
            pl.BlockSpec((bm, kdim), lambda i: (jnp.minimum(i, last), 0)),
            pl.BlockSpec((n, kdim), lambda i: (0, 0)),
            pl.BlockSpec((1, n), lambda i: (0, 0)),
        ],
        out_specs=[
            pl.BlockSpec((bm, n), lambda i: (jnp.maximum(i - 1, 0), 0)),
            pl.BlockSpec((bm, n), lambda i: (jnp.minimum(i, last), 0)),
        ],
        out_shape=[
            jax.ShapeDtypeStruct((m, n), jnp.float32),
            jax.ShapeDtypeStruct((m, n), jnp.float32),
        ],
        scratch_shapes=[
            pltpu.VMEM((bm, n), jnp.int32),
            pltpu.VMEM((bm, kdim), jnp.float32),
        ],
        compiler_params=pltpu.CompilerParams(
            dimension_semantics=("arbitrary",)),
    )(x, W.astype(jnp.bfloat16), b.reshape(1, n))
    return (masked, scores)
